# baseline (speedup 1.0000x reference)
import functools

import jax
import jax.numpy as jnp
from jax import lax
from jax.experimental import pallas as pl
from jax.experimental.pallas import tpu as pltpu
from jax.experimental.pallas import tpu_sc as plsc

_D = 128

_mesh = plsc.ScalarSubcoreMesh(axis_name="c", num_cores=1)


@functools.partial(
    pl.kernel,
    mesh=_mesh,
    out_type=jax.ShapeDtypeStruct((4, _D), jnp.float32),
    scratch_types=[pltpu.SemaphoreType.DMA],
)
def _gather_rows(x3d_hbm, out_hbm, sem):
    copies = [
        pltpu.make_async_copy(
            x3d_hbm.at[pl.ds(0, 2), j], out_hbm.at[pl.ds(2 * j, 2)], sem
        )
        for j in (0, 1)
    ]
    for cp in copies:
        cp.start()
    for cp in copies:
        cp.wait()


def kernel(x):
    return _gather_rows(x.reshape(50000, 2, _D)).reshape(2, 2, _D)


# speedup vs baseline: 1.0032x; 1.0032x over previous
import functools

import jax
import jax.numpy as jnp
from jax import lax
from jax.experimental import pallas as pl
from jax.experimental.pallas import tpu as pltpu
from jax.experimental.pallas import tpu_sc as plsc

_PERM = (0, 2, 1, 3)
_D = 128

_mesh = plsc.ScalarSubcoreMesh(axis_name="c", num_cores=1)


@functools.partial(
    pl.kernel,
    mesh=_mesh,
    out_type=jax.ShapeDtypeStruct((2, 2, _D), jnp.float32),
    scratch_types=[pltpu.SemaphoreType.DMA],
)
def _gather_rows(x_hbm, out_hbm, sem):
    copies = [
        pltpu.make_async_copy(x_hbm.at[src], out_hbm.at[i // 2, i % 2], sem)
        for i, src in enumerate(_PERM)
    ]
    for cp in copies:
        cp.start()
    for cp in copies:
        cp.wait()


def kernel(x):
    return _gather_rows(x)
